# baseline (speedup 1.0000x reference)
import functools

import jax
import jax.numpy as jnp
from jax import lax
from jax.experimental import pallas as pl
from jax.experimental.pallas import tpu as pltpu
from jax.experimental.pallas import tpu_sc as plsc

_N_USERS = 10000
_N_ENT = 40000
_N_NODES = 50000
_C = 128
_NCH = 4
_CW = 32
_T = 16
_B = 512
_IB = 128
_MESH = dict(core_axis_name="c", subcore_axis_name="s")
_SC_PARAMS = pltpu.CompilerParams(use_tc_tiling_on_sc=False)


def _qk_body(n_ref, wq_ref, wk_ref, q_ref, k_ref):
    x = n_ref[...]
    q_ref[...] = jnp.dot(x, wq_ref[...], preferred_element_type=jnp.float32)
    k_ref[...] = jnp.dot(x, wk_ref[...], preferred_element_type=jnp.float32)


def _qk(node, wq, wk):
    n = node.shape[0]
    bn = 2000
    return pl.pallas_call(
        _qk_body,
        grid=(n // bn,),
        in_specs=[
            pl.BlockSpec((bn, _C), lambda i: (i, 0)),
            pl.BlockSpec((_C, _C), lambda i: (0, 0)),
            pl.BlockSpec((_C, _C), lambda i: (0, 0)),
        ],
        out_specs=[
            pl.BlockSpec((bn, _C), lambda i: (i, 0)),
            pl.BlockSpec((bn, _C), lambda i: (i, 0)),
        ],
        out_shape=[jax.ShapeDtypeStruct((n, _C), jnp.float32)] * 2,
    )(node, wq, wk)


def _chunk_body(x_ref, o_ref):
    x = x_ref[...]
    for c in range(_NCH):
        o_ref[c] = x[:, c * _CW:(c + 1) * _CW]


def _chunkify(x):
    n = x.shape[0]
    bn = 2000
    return pl.pallas_call(
        _chunk_body,
        grid=(n // bn,),
        in_specs=[pl.BlockSpec((bn, _C), lambda i: (i, 0))],
        out_specs=pl.BlockSpec((_NCH, bn, _CW), lambda i: (0, i, 0)),
        out_shape=jax.ShapeDtypeStruct((_NCH, n, _CW), jnp.float32),
    )(x)


def _unchunk_body(x_ref, o_ref):
    for c in range(_NCH):
        o_ref[:, c * _CW:(c + 1) * _CW] = x_ref[c]


def _unchunk(x):
    n = x.shape[1]
    bn = 2000
    return pl.pallas_call(
        _unchunk_body,
        grid=(n // bn,),
        in_specs=[pl.BlockSpec((_NCH, bn, _CW), lambda i: (0, i, 0))],
        out_specs=pl.BlockSpec((bn, _C), lambda i: (i, 0)),
        out_shape=jax.ShapeDtypeStruct((n, _C), jnp.float32),
    )(x)


def _post(sums, cnt_rows, res_in, *, mean, attn, want_mean):
    n = sums.shape[1]
    bn = 2000
    grid = (n // bn,)

    def body(*refs):
        if mean:
            s_ref, c_ref, r_ref = refs[:3]
            orefs = refs[3:]
        else:
            s_ref, r_ref = refs[:2]
            orefs = refs[2:]
        s = s_ref[...]
        if mean:
            crow = c_ref[...]
            inv = 1.0 / jnp.maximum(crow[:, 0:1], 1.0)
            if attn:
                inv = inv / jnp.maximum(crow[:, 1:2], 1e-12)
            s = s * inv[None]
        nrm2 = jnp.sum(s * s, axis=(0, 2), keepdims=True)
        rinv = 1.0 / jnp.maximum(jnp.sqrt(nrm2), 1e-12)
        normed = s * rinv
        if want_mean:
            orefs[0][...] = s
            orefs = orefs[1:]
        orefs[0][...] = normed
        orefs[1][...] = r_ref[...] + normed

    in_specs = [pl.BlockSpec((_NCH, bn, _CW), lambda i: (0, i, 0))]
    args = [sums]
    if mean:
        in_specs.append(pl.BlockSpec((bn, 16), lambda i: (i, 0)))
        args.append(cnt_rows)
    in_specs.append(pl.BlockSpec((_NCH, bn, _CW), lambda i: (0, i, 0)))
    args.append(res_in)
    n_out = 3 if want_mean else 2
    out_specs = [pl.BlockSpec((_NCH, bn, _CW), lambda i: (0, i, 0))] * n_out
    out_shape = [jax.ShapeDtypeStruct((_NCH, n, _CW), jnp.float32)] * n_out
    outs = pl.pallas_call(
        body, grid=grid, in_specs=in_specs, out_specs=out_specs,
        out_shape=out_shape,
    )(*args)
    if want_mean:
        return outs[0], outs[1], outs[2]
    return None, outs[0], outs[1]


def _hsum16(v):
    for k in (8, 4, 2, 1):
        idx = jnp.bitwise_xor(lax.iota(jnp.int32, 16), k)
        v = v + v.at[idx].get(mode="promise_in_bounds")
    return v[0]

def _attn_kernel(e_pad):
    pw = e_pad // (2 * _T)
    nb = pw // _IB

    @functools.partial(
        pl.kernel,
        out_type=jax.ShapeDtypeStruct((e_pad,), jnp.float32),
        mesh=plsc.VectorSubcoreMesh(**_MESH),
        scratch_types=[
            pltpu.VMEM((_IB,), jnp.int32),
            pltpu.VMEM((_IB,), jnp.int32),
            pltpu.VMEM((_IB, _C), jnp.float32),
            pltpu.VMEM((_IB, _C), jnp.float32),
            pltpu.VMEM((_IB,), jnp.float32),
            pltpu.SemaphoreType.DMA,
        ],
        compiler_params=_SC_PARAMS,
    )
    def kern(q_hbm, k_hbm, h_hbm, t_hbm, e_hbm, hi_v, ti_v, qv, kv, ev, sem):
        c = lax.axis_index("c")
        s = lax.axis_index("s")
        base = (s * 2 + c) * pw

        def batch(b, _):
            off = base + b * _IB
            pltpu.sync_copy(h_hbm.at[pl.ds(off, _IB)], hi_v)
            pltpu.sync_copy(t_hbm.at[pl.ds(off, _IB)], ti_v)
            cp1 = pltpu.async_copy(q_hbm.at[hi_v], qv, sem)
            cp2 = pltpu.async_copy(k_hbm.at[ti_v], kv, sem)
            cp1.wait()
            cp2.wait()

            @plsc.parallel_loop(0, _IB // 16, unroll=4)
            def _group(g):
                def edge(i, vec):
                    e = g * 16 + i
                    acc = qv[e, pl.ds(0, 16)] * kv[e, pl.ds(0, 16)]
                    for cc in range(1, 8):
                        acc = acc + qv[e, pl.ds(cc * 16, 16)] * kv[e, pl.ds(cc * 16, 16)]
                    sdot = _hsum16(acc)
                    return jnp.where(lax.iota(jnp.int32, 16) == i, sdot, vec)

                vec = lax.fori_loop(0, 16, edge, jnp.zeros((16,), jnp.float32))
                ev[pl.ds(g * 16, 16)] = jnp.exp(vec * (1.0 / 24.0))
            pltpu.sync_copy(ev, e_hbm.at[pl.ds(off, _IB)])
            return 0

        lax.fori_loop(0, nb, batch, 0)

    return kern


def _counts_kernel(e_ex_pad, e_kg_pad):
    acc_rows = _N_NODES + 16
    nb_ex = e_ex_pad // _T // _B
    nb_kg = e_kg_pad // _T // _B

    @functools.partial(
        pl.kernel,
        out_type=(
            jax.ShapeDtypeStruct((_N_NODES, 16), jnp.float32),
            jax.ShapeDtypeStruct((_N_ENT, 16), jnp.float32),
        ),
        mesh=plsc.VectorSubcoreMesh(**_MESH),
        scratch_types=[
            pltpu.VMEM((_B,), jnp.int32),
            pltpu.VMEM((_B // _IB, _IB), jnp.int32),
            pltpu.VMEM((_B,), jnp.float32),
            pltpu.VMEM((_B, 16), jnp.float32),
            pltpu.VMEM((_IB, 16), jnp.float32),
            pltpu.VMEM_SHARED((acc_rows, 16), jnp.float32),
            pltpu.SemaphoreType.DMA,
            pltpu.SemaphoreType.DMA,
        ],
        compiler_params=_SC_PARAMS,
    )
    def kern(hh_hbm, e_hbm, h_hbm, oex_hbm, okg_hbm,
             sr_v, si_v, e_v, rows_v, tmp_v, acc, sem, sem2):
        def repack_idx():
            for jj in range(_B // _IB):
                for l in range(_IB // 16):
                    si_v[jj, pl.ds(l * 16, 16)] = sr_v[pl.ds(jj * _IB + l * 16, 16)]
        c = lax.axis_index("c")
        s = lax.axis_index("s")

        def fill_tmp(j, _):
            tmp_v[j, pl.ds(0, 16)] = jnp.zeros((16,), jnp.float32)
            return 0

        lax.fori_loop(0, _IB, fill_tmp, 0)

        zr = acc_rows // _T
        zf, zrem = zr // _IB, zr % _IB
        for zb in range(zf):
            pltpu.sync_copy(tmp_v, acc.at[pl.ds(s * zr + zb * _IB, _IB)])
        if zrem:
            pltpu.sync_copy(tmp_v.at[pl.ds(0, zrem)],
                            acc.at[pl.ds(s * zr + zf * _IB, zrem)])
        plsc.subcore_barrier()

        ones16 = jnp.where(lax.iota(jnp.int32, 16) == 0, 1.0, 0.0).astype(jnp.float32)

        def fill_ones(i, _):
            rows_v[i, pl.ds(0, 16)] = ones16
            return 0

        def ex_phase():
            pt = e_ex_pad // _T

            def batch(b, _):
                off = s * pt + b * _B
                pltpu.sync_copy(hh_hbm.at[pl.ds(off, _B)], sr_v)
                pltpu.sync_copy(e_hbm.at[pl.ds(off, _B)], e_v)
                repack_idx()

                @plsc.parallel_loop(0, _B // 16, unroll=4)
                def _group(g):
                    gb = g * 16
                    e16 = e_v[pl.ds(gb, 16)]
                    for i in range(16):
                        rows_v[gb + i, pl.ds(0, 16)] = ones16 + jnp.where(
                            lax.iota(jnp.int32, 16) == 1, e16[i], 0.0)
                cps = [
                    pltpu.async_copy(rows_v.at[pl.ds(j * _IB, _IB)],
                                     acc.at[si_v.at[j]], sem2, add=True)
                    for j in range(_B // _IB)
                ]
                for cp in cps:
                    cp.wait()
                return 0

            lax.fori_loop(0, nb_ex, batch, 0)

        def kg_phase():
            pt = e_kg_pad // _T
            lax.fori_loop(0, _B, fill_ones, 0)

            def batch(b, _):
                off = s * pt + b * _B
                pltpu.sync_copy(h_hbm.at[pl.ds(off, _B)], sr_v)
                repack_idx()
                cps = [
                    pltpu.async_copy(rows_v.at[pl.ds(j * _IB, _IB)],
                                     acc.at[si_v.at[j]], sem2, add=True)
                    for j in range(_B // _IB)
                ]
                for cp in cps:
                    cp.wait()
                return 0

            lax.fori_loop(0, nb_kg, batch, 0)

        pl.when(c == 0)(ex_phase)
        pl.when(c == 1)(kg_phase)
        plsc.subcore_barrier()

        def drain(n_dst, out_hbm):
            nblk, rem = n_dst // _IB, n_dst % _IB
            nper = -(-nblk // _T)

            def dblk(j, _):
                b = j * _T + s

                @pl.when(b < nblk)
                def _():
                    pltpu.sync_copy(acc.at[pl.ds(b * _IB, _IB)], tmp_v)
                    pltpu.sync_copy(tmp_v, out_hbm.at[pl.ds(b * _IB, _IB)])
                return 0

            lax.fori_loop(0, nper, dblk, 0)
            if rem:
                @pl.when(s == 0)
                def _():
                    pltpu.sync_copy(acc.at[pl.ds(nblk * _IB, rem)],
                                    tmp_v.at[pl.ds(0, rem)])
                    pltpu.sync_copy(tmp_v.at[pl.ds(0, rem)],
                                    out_hbm.at[pl.ds(nblk * _IB, rem)])

        pl.when(c == 0)(lambda: drain(_N_NODES, oex_hbm))
        pl.when(c == 1)(lambda: drain(_N_ENT, okg_hbm))

    return kern


def _rowscatter_kernel(e_pad, n_src, n_dst, mode, n_wrows=0):
    pt = e_pad // _T
    nb = pt // _B
    acc_rows = n_dst + 16
    zr = acc_rows // _T
    dr = n_dst // _T

    scratch = [
        pltpu.VMEM((_B,), jnp.int32),
        pltpu.VMEM((_B,), jnp.int32),
        pltpu.VMEM((_B // _IB, _IB), jnp.int32),
        pltpu.VMEM((_B // _IB, _IB), jnp.int32),
        pltpu.VMEM((_B, _CW), jnp.float32),
        pltpu.VMEM((_IB, _CW), jnp.float32),
        pltpu.VMEM_SHARED((acc_rows, _CW), jnp.float32),
        pltpu.SemaphoreType.DMA,
        pltpu.SemaphoreType.DMA,
    ]
    if mode in ("kg", "ex"):
        scratch.append(pltpu.VMEM((_NCH * n_wrows, _CW), jnp.float32))
        scratch.append(pltpu.VMEM((_B,), jnp.int32))
    if mode in ("ex", "usr"):
        scratch.append(pltpu.VMEM((_B,), jnp.float32))

    def body(refs):
        if mode == "kg":
            (src, gi_hbm, si_hbm, ty_hbm, w_hbm, out,
             gr_v, sr_v, gi_v, si_v, rows_v, tmp_v, acc, sem, sem2, w_v, ty_v) = refs
        elif mode == "ex":
            (src, gi_hbm, si_hbm, ty_hbm, w_hbm, sc_hbm, out,
             gr_v, sr_v, gi_v, si_v, rows_v, tmp_v, acc, sem, sem2, w_v, ty_v, sc_v) = refs
        else:
            (src, gi_hbm, si_hbm, sc_hbm, out,
             gr_v, sr_v, gi_v, si_v, rows_v, tmp_v, acc, sem, sem2, sc_v) = refs

        c = lax.axis_index("c")
        s = lax.axis_index("s")
        ebase = s * pt

        def fill_tmp(j, _):
            tmp_v[j, pl.ds(0, 16)] = jnp.zeros((16,), jnp.float32)
            tmp_v[j, pl.ds(16, 16)] = jnp.zeros((16,), jnp.float32)
            return 0

        lax.fori_loop(0, _IB, fill_tmp, 0)
        if mode in ("kg", "ex"):
            pltpu.sync_copy(w_hbm, w_v)

        for cc in range(2):
            chunk = c * 2 + cc

            zf, zrem = zr // _IB, zr % _IB
            for zb in range(zf):
                pltpu.sync_copy(tmp_v, acc.at[pl.ds(s * zr + zb * _IB, _IB)])
            if zrem:
                pltpu.sync_copy(tmp_v.at[pl.ds(0, zrem)],
                                acc.at[pl.ds(s * zr + zf * _IB, zrem)])
            plsc.subcore_barrier()

            goff = chunk * n_src

            def batch(b, _):
                off = ebase + b * _B
                pltpu.sync_copy(gi_hbm.at[pl.ds(off, _B)], gr_v)
                pltpu.sync_copy(si_hbm.at[pl.ds(off, _B)], sr_v)
                if mode in ("kg", "ex"):
                    pltpu.sync_copy(ty_hbm.at[pl.ds(off, _B)], ty_v)
                if mode in ("ex", "usr"):
                    pltpu.sync_copy(sc_hbm.at[pl.ds(off, _B)], sc_v)
                for jj in range(_B // _IB):
                    for l in range(_IB // 16):
                        fl = jj * _IB + l * 16
                        gi_v[jj, pl.ds(l * 16, 16)] = gr_v[pl.ds(fl, 16)] + goff
                        si_v[jj, pl.ds(l * 16, 16)] = sr_v[pl.ds(fl, 16)]
                cps = [
                    pltpu.async_copy(src.at[gi_v.at[j]],
                                     rows_v.at[pl.ds(j * _IB, _IB)], sem)
                    for j in range(_B // _IB)
                ]
                for cp in cps:
                    cp.wait()

                @plsc.parallel_loop(0, _B // 16, unroll=4)
                def _group(g):
                    gb = g * 16
                    if mode in ("kg", "ex"):
                        ty16 = ty_v[pl.ds(gb, 16)] + chunk * n_wrows
                    if mode in ("ex", "usr"):
                        sc16 = sc_v[pl.ds(gb, 16)]
                    for i in range(16):
                        e = gb + i
                        r0 = rows_v[e, pl.ds(0, 16)]
                        r1 = rows_v[e, pl.ds(16, 16)]
                        if mode in ("kg", "ex"):
                            ty = ty16[i]
                            r0 = r0 * w_v[ty, pl.ds(0, 16)]
                            r1 = r1 * w_v[ty, pl.ds(16, 16)]
                        if mode in ("ex", "usr"):
                            sc = sc16[i]
                            r0 = r0 * sc
                            r1 = r1 * sc
                        rows_v[e, pl.ds(0, 16)] = r0
                        rows_v[e, pl.ds(16, 16)] = r1
                cps = [
                    pltpu.async_copy(rows_v.at[pl.ds(j * _IB, _IB)],
                                     acc.at[si_v.at[j]], sem2, add=True)
                    for j in range(_B // _IB)
                ]
                for cp in cps:
                    cp.wait()
                return 0

            lax.fori_loop(0, nb, batch, 0)
            plsc.subcore_barrier()

            nblk, rem = n_dst // _IB, n_dst % _IB
            nper = -(-nblk // _T)

            def dblk(j, _):
                b = j * _T + s

                @pl.when(b < nblk)
                def _():
                    pltpu.sync_copy(acc.at[pl.ds(b * _IB, _IB)], tmp_v)
                    pltpu.sync_copy(tmp_v, out.at[pl.ds(chunk * n_dst + b * _IB, _IB)])
                return 0

            lax.fori_loop(0, nper, dblk, 0)
            if rem:
                @pl.when(s == 0)
                def _():
                    pltpu.sync_copy(acc.at[pl.ds(nblk * _IB, rem)],
                                    tmp_v.at[pl.ds(0, rem)])
                    pltpu.sync_copy(tmp_v.at[pl.ds(0, rem)],
                                    out.at[pl.ds(chunk * n_dst + nblk * _IB, rem)])
            lax.fori_loop(0, _IB, fill_tmp, 0)
            plsc.subcore_barrier()

    def kern_fn(*refs):
        body(refs)

    return functools.partial(
        pl.kernel,
        out_type=jax.ShapeDtypeStruct((_NCH * n_dst, _CW), jnp.float32),
        mesh=plsc.VectorSubcoreMesh(**_MESH),
        scratch_types=scratch,
        compiler_params=_SC_PARAMS,
    )(kern_fn)


def _pad_to(x, n, val):
    if n > x.shape[0]:
        return jnp.pad(x, (0, n - x.shape[0]), constant_values=val)
    return x


def _padlen(e):
    q = _T * _B
    return -(-e // q) * q


def kernel(user_emb, entity_emb, interact_indices, interact_values,
           edge_index, edge_type, extra_edge_index, extra_edge_type,
           weight, extra_weight, W_Q, W_K):
    f32 = jnp.float32
    node_emb = jnp.concatenate([user_emb, entity_emb], axis=0)

    e_kg = edge_index.shape[1]
    e_ex = extra_edge_index.shape[1]
    nnz = interact_values.shape[0]
    ekg, eex, enz = _padlen(e_kg), _padlen(e_ex), _padlen(nnz)

    kg_g = _pad_to(edge_index[1].astype(jnp.int32), ekg, 0)
    kg_s = _pad_to(edge_index[0].astype(jnp.int32), ekg, _N_ENT)
    kg_t = _pad_to((edge_type - 1).astype(jnp.int32), ekg, 0)
    ex_g = _pad_to(extra_edge_index[1].astype(jnp.int32), eex, 0)
    ex_s = _pad_to(extra_edge_index[0].astype(jnp.int32), eex, _N_NODES)
    ex_h = _pad_to(extra_edge_index[0].astype(jnp.int32), eex, 0)
    ex_t = _pad_to(extra_edge_type.astype(jnp.int32), eex, 0)
    us_g = _pad_to(interact_indices[1].astype(jnp.int32), enz, 0)
    us_s = _pad_to(interact_indices[0].astype(jnp.int32), enz, _N_USERS)
    us_v = _pad_to(interact_values.astype(f32), enz, 0.0)

    wkg = weight.reshape(10, _NCH, _CW).transpose(1, 0, 2).reshape(-1, _CW)
    wex = extra_weight.reshape(8, _NCH, _CW).transpose(1, 0, 2).reshape(-1, _CW)

    q, k = _qk(node_emb, W_Q, W_K)
    e_arr = _attn_kernel(eex)(q, k, ex_h, ex_g)
    cnt_ex, cnt_kg = _counts_kernel(eex, ekg)(ex_s, e_arr, kg_s)

    entc = _chunkify(entity_emb)
    nodc = _chunkify(node_emb)
    usrc = _chunkify(user_emb)
    ent_res, nod_res, usr_res = entc, nodc, usrc

    kg_scatter = _rowscatter_kernel(ekg, _N_ENT, _N_ENT, "kg", 10)
    ex_scatter = _rowscatter_kernel(eex, _N_NODES, _N_NODES, "ex", 8)
    us_scatter = _rowscatter_kernel(enz, _N_ENT, _N_USERS, "usr")

    ent_g = entc.reshape(_NCH * _N_ENT, _CW)
    nod_g = nodc.reshape(_NCH * _N_NODES, _CW)
    for _ in range(2):
        sums_kg = kg_scatter(ent_g, kg_g, kg_s, kg_t, wkg)
        ent_mean, ent_norm, ent_res = _post(
            sums_kg.reshape(_NCH, _N_ENT, _CW), cnt_kg, ent_res,
            mean=True, attn=False, want_mean=True)
        sums_ex = ex_scatter(nod_g, ex_g, ex_s, ex_t, wex, e_arr)
        _, nod_norm, nod_res = _post(
            sums_ex.reshape(_NCH, _N_NODES, _CW), cnt_ex, nod_res,
            mean=True, attn=True, want_mean=False)
        sums_us = us_scatter(ent_mean.reshape(_NCH * _N_ENT, _CW),
                             us_g, us_s, us_v)
        _, _, usr_res = _post(
            sums_us.reshape(_NCH, _N_USERS, _CW), None, usr_res,
            mean=False, attn=False, want_mean=False)
        ent_g = ent_norm.reshape(_NCH * _N_ENT, _CW)
        nod_g = nod_norm.reshape(_NCH * _N_NODES, _CW)

    return _unchunk(ent_res), _unchunk(usr_res), _unchunk(nod_res)


# speedup vs baseline: 1.2913x; 1.2913x over previous
import functools

import jax
import jax.numpy as jnp
from jax import lax
from jax.experimental import pallas as pl
from jax.experimental.pallas import tpu as pltpu
from jax.experimental.pallas import tpu_sc as plsc

_N_USERS = 10000
_N_ENT = 40000
_N_NODES = 50000
_C = 128
_NCH = 4
_CW = 32
_T = 16
_B = 512
_IB = 128
_SB = 256
_NI = _SB // _IB
_MESH = dict(core_axis_name="c", subcore_axis_name="s")
_SC_PARAMS = pltpu.CompilerParams(use_tc_tiling_on_sc=False)


def _qk_body(n_ref, wq_ref, wk_ref, q_ref, k_ref):
    x = n_ref[...]
    q_ref[...] = jnp.dot(x, wq_ref[...], preferred_element_type=jnp.float32)
    k_ref[...] = jnp.dot(x, wk_ref[...], preferred_element_type=jnp.float32)


def _qk(node, wq, wk):
    n = node.shape[0]
    bn = 2000
    return pl.pallas_call(
        _qk_body,
        grid=(n // bn,),
        in_specs=[
            pl.BlockSpec((bn, _C), lambda i: (i, 0)),
            pl.BlockSpec((_C, _C), lambda i: (0, 0)),
            pl.BlockSpec((_C, _C), lambda i: (0, 0)),
        ],
        out_specs=[
            pl.BlockSpec((bn, _C), lambda i: (i, 0)),
            pl.BlockSpec((bn, _C), lambda i: (i, 0)),
        ],
        out_shape=[jax.ShapeDtypeStruct((n, _C), jnp.float32)] * 2,
    )(node, wq, wk)


def _chunk_body(x_ref, o_ref):
    x = x_ref[...]
    for c in range(_NCH):
        o_ref[c] = x[:, c * _CW:(c + 1) * _CW]


def _chunkify(x):
    n = x.shape[0]
    bn = 2000
    return pl.pallas_call(
        _chunk_body,
        grid=(n // bn,),
        in_specs=[pl.BlockSpec((bn, _C), lambda i: (i, 0))],
        out_specs=pl.BlockSpec((_NCH, bn, _CW), lambda i: (0, i, 0)),
        out_shape=jax.ShapeDtypeStruct((_NCH, n, _CW), jnp.float32),
    )(x)


def _unchunk_body(x_ref, o_ref):
    for c in range(_NCH):
        o_ref[:, c * _CW:(c + 1) * _CW] = x_ref[c]


def _unchunk(x):
    n = x.shape[1]
    bn = 2000
    return pl.pallas_call(
        _unchunk_body,
        grid=(n // bn,),
        in_specs=[pl.BlockSpec((_NCH, bn, _CW), lambda i: (0, i, 0))],
        out_specs=pl.BlockSpec((bn, _C), lambda i: (i, 0)),
        out_shape=jax.ShapeDtypeStruct((n, _C), jnp.float32),
    )(x)


def _post(sums, cnt_rows, res_in, *, mean, attn, want_mean):
    n = sums.shape[1]
    bn = 2000
    grid = (n // bn,)

    def body(*refs):
        if mean:
            s_ref, c_ref, r_ref = refs[:3]
            orefs = refs[3:]
        else:
            s_ref, r_ref = refs[:2]
            orefs = refs[2:]
        s = s_ref[...]
        if mean:
            crow = c_ref[...]
            inv = 1.0 / jnp.maximum(crow[:, 0:1], 1.0)
            if attn:
                inv = inv / jnp.maximum(crow[:, 1:2], 1e-12)
            s = s * inv[None]
        nrm2 = jnp.sum(s * s, axis=(0, 2), keepdims=True)
        rinv = 1.0 / jnp.maximum(jnp.sqrt(nrm2), 1e-12)
        normed = s * rinv
        if want_mean:
            orefs[0][...] = s
            orefs = orefs[1:]
        orefs[0][...] = normed
        orefs[1][...] = r_ref[...] + normed

    in_specs = [pl.BlockSpec((_NCH, bn, _CW), lambda i: (0, i, 0))]
    args = [sums]
    if mean:
        in_specs.append(pl.BlockSpec((bn, 16), lambda i: (i, 0)))
        args.append(cnt_rows)
    in_specs.append(pl.BlockSpec((_NCH, bn, _CW), lambda i: (0, i, 0)))
    args.append(res_in)
    n_out = 3 if want_mean else 2
    out_specs = [pl.BlockSpec((_NCH, bn, _CW), lambda i: (0, i, 0))] * n_out
    out_shape = [jax.ShapeDtypeStruct((_NCH, n, _CW), jnp.float32)] * n_out
    outs = pl.pallas_call(
        body, grid=grid, in_specs=in_specs, out_specs=out_specs,
        out_shape=out_shape,
    )(*args)
    if want_mean:
        return outs[0], outs[1], outs[2]
    return None, outs[0], outs[1]


def _hsum16(v):
    for k in (8, 4, 2, 1):
        idx = jnp.bitwise_xor(lax.iota(jnp.int32, 16), k)
        v = v + v.at[idx].get(mode="promise_in_bounds")
    return v[0]

def _attn_kernel(e_pad):
    pw = e_pad // (2 * _T)
    nb = pw // _IB

    @functools.partial(
        pl.kernel,
        out_type=jax.ShapeDtypeStruct((e_pad,), jnp.float32),
        mesh=plsc.VectorSubcoreMesh(**_MESH),
        scratch_types=[
            pltpu.VMEM((_IB,), jnp.int32),
            pltpu.VMEM((_IB,), jnp.int32),
            pltpu.VMEM((_IB, _C), jnp.float32),
            pltpu.VMEM((_IB, _C), jnp.float32),
            pltpu.VMEM((_IB,), jnp.float32),
            pltpu.SemaphoreType.DMA,
        ],
        compiler_params=_SC_PARAMS,
    )
    def kern(q_hbm, k_hbm, h_hbm, t_hbm, e_hbm, hi_v, ti_v, qv, kv, ev, sem):
        c = lax.axis_index("c")
        s = lax.axis_index("s")
        base = (s * 2 + c) * pw

        def batch(b, _):
            off = base + b * _IB
            pltpu.sync_copy(h_hbm.at[pl.ds(off, _IB)], hi_v)
            pltpu.sync_copy(t_hbm.at[pl.ds(off, _IB)], ti_v)
            cp1 = pltpu.async_copy(q_hbm.at[hi_v], qv, sem)
            cp2 = pltpu.async_copy(k_hbm.at[ti_v], kv, sem)
            cp1.wait()
            cp2.wait()

            @plsc.parallel_loop(0, _IB // 16, unroll=2)
            def _group(g):
                def edge(i, vec):
                    e = g * 16 + i
                    acc = qv[e, pl.ds(0, 16)] * kv[e, pl.ds(0, 16)]
                    for cc in range(1, 8):
                        acc = acc + qv[e, pl.ds(cc * 16, 16)] * kv[e, pl.ds(cc * 16, 16)]
                    sdot = _hsum16(acc)
                    return jnp.where(lax.iota(jnp.int32, 16) == i, sdot, vec)

                vec = lax.fori_loop(0, 16, edge, jnp.zeros((16,), jnp.float32))
                ev[pl.ds(g * 16, 16)] = jnp.exp(vec * (1.0 / 24.0))
            pltpu.sync_copy(ev, e_hbm.at[pl.ds(off, _IB)])
            return 0

        lax.fori_loop(0, nb, batch, 0)

    return kern


def _counts_kernel(e_ex_pad, e_kg_pad):
    acc_rows = _N_NODES + 16
    nb_ex = e_ex_pad // _T // _B
    nb_kg = e_kg_pad // _T // _B

    @functools.partial(
        pl.kernel,
        out_type=(
            jax.ShapeDtypeStruct((_N_NODES, 16), jnp.float32),
            jax.ShapeDtypeStruct((_N_ENT, 16), jnp.float32),
        ),
        mesh=plsc.VectorSubcoreMesh(**_MESH),
        scratch_types=[
            pltpu.VMEM((_B,), jnp.int32),
            pltpu.VMEM((_B // _IB, _IB), jnp.int32),
            pltpu.VMEM((_B,), jnp.float32),
            pltpu.VMEM((_B, 16), jnp.float32),
            pltpu.VMEM((_IB, 16), jnp.float32),
            pltpu.VMEM_SHARED((acc_rows, 16), jnp.float32),
            pltpu.SemaphoreType.DMA,
            pltpu.SemaphoreType.DMA,
        ],
        compiler_params=_SC_PARAMS,
    )
    def kern(hh_hbm, e_hbm, h_hbm, oex_hbm, okg_hbm,
             sr_v, si_v, e_v, rows_v, tmp_v, acc, sem, sem2):
        def repack_idx():
            for jj in range(_B // _IB):
                for l in range(_IB // 16):
                    si_v[jj, pl.ds(l * 16, 16)] = sr_v[pl.ds(jj * _IB + l * 16, 16)]
        c = lax.axis_index("c")
        s = lax.axis_index("s")

        def fill_tmp(j, _):
            tmp_v[j, pl.ds(0, 16)] = jnp.zeros((16,), jnp.float32)
            return 0

        lax.fori_loop(0, _IB, fill_tmp, 0)

        zr = acc_rows // _T
        zf, zrem = zr // _IB, zr % _IB
        for zb in range(zf):
            pltpu.sync_copy(tmp_v, acc.at[pl.ds(s * zr + zb * _IB, _IB)])
        if zrem:
            pltpu.sync_copy(tmp_v.at[pl.ds(0, zrem)],
                            acc.at[pl.ds(s * zr + zf * _IB, zrem)])
        plsc.subcore_barrier()

        ones16 = jnp.where(lax.iota(jnp.int32, 16) == 0, 1.0, 0.0).astype(jnp.float32)

        def fill_ones(i, _):
            rows_v[i, pl.ds(0, 16)] = ones16
            return 0

        def ex_phase():
            pt = e_ex_pad // _T

            def batch(b, _):
                off = s * pt + b * _B
                pltpu.sync_copy(hh_hbm.at[pl.ds(off, _B)], sr_v)
                pltpu.sync_copy(e_hbm.at[pl.ds(off, _B)], e_v)
                repack_idx()

                @plsc.parallel_loop(0, _B // 16, unroll=2)
                def _group(g):
                    gb = g * 16
                    e16 = e_v[pl.ds(gb, 16)]
                    for i in range(16):
                        rows_v[gb + i, pl.ds(0, 16)] = ones16 + jnp.where(
                            lax.iota(jnp.int32, 16) == 1, e16[i], 0.0)
                cps = [
                    pltpu.async_copy(rows_v.at[pl.ds(j * _IB, _IB)],
                                     acc.at[si_v.at[j]], sem2, add=True)
                    for j in range(_B // _IB)
                ]
                for cp in cps:
                    cp.wait()
                return 0

            lax.fori_loop(0, nb_ex, batch, 0)

        def kg_phase():
            pt = e_kg_pad // _T
            lax.fori_loop(0, _B, fill_ones, 0)

            def batch(b, _):
                off = s * pt + b * _B
                pltpu.sync_copy(h_hbm.at[pl.ds(off, _B)], sr_v)
                repack_idx()
                cps = [
                    pltpu.async_copy(rows_v.at[pl.ds(j * _IB, _IB)],
                                     acc.at[si_v.at[j]], sem2, add=True)
                    for j in range(_B // _IB)
                ]
                for cp in cps:
                    cp.wait()
                return 0

            lax.fori_loop(0, nb_kg, batch, 0)

        pl.when(c == 0)(ex_phase)
        pl.when(c == 1)(kg_phase)
        plsc.subcore_barrier()

        def drain(n_dst, out_hbm):
            nblk, rem = n_dst // _IB, n_dst % _IB
            nper = -(-nblk // _T)

            def dblk(j, _):
                b = j * _T + s

                @pl.when(b < nblk)
                def _():
                    pltpu.sync_copy(acc.at[pl.ds(b * _IB, _IB)], tmp_v)
                    pltpu.sync_copy(tmp_v, out_hbm.at[pl.ds(b * _IB, _IB)])
                return 0

            lax.fori_loop(0, nper, dblk, 0)
            if rem:
                @pl.when(s == 0)
                def _():
                    pltpu.sync_copy(acc.at[pl.ds(nblk * _IB, rem)],
                                    tmp_v.at[pl.ds(0, rem)])
                    pltpu.sync_copy(tmp_v.at[pl.ds(0, rem)],
                                    out_hbm.at[pl.ds(nblk * _IB, rem)])

        pl.when(c == 0)(lambda: drain(_N_NODES, oex_hbm))
        pl.when(c == 1)(lambda: drain(_N_ENT, okg_hbm))

    return kern


def _rowscatter_kernel(e_pad, n_src, n_dst, mode, n_wrows=0):
    pt = e_pad // _T
    nb2 = pt // _SB
    assert nb2 % 2 == 0
    acc_rows = n_dst + 16
    zr = acc_rows // _T
    _ZB = 64

    scratch = [
        pltpu.VMEM((3 * _SB,), jnp.int32),
        pltpu.VMEM((3 * _SB,), jnp.int32),
        pltpu.VMEM((_NI, _IB), jnp.int32),
        pltpu.VMEM((_NI, _IB), jnp.int32),
        pltpu.VMEM((_NI, _IB), jnp.int32),
        pltpu.VMEM((_NI, _IB), jnp.int32),
        pltpu.VMEM((_NI, _IB), jnp.int32),
        pltpu.VMEM((_NI, _IB), jnp.int32),
        pltpu.VMEM((_SB, _CW), jnp.float32),
        pltpu.VMEM((_SB, _CW), jnp.float32),
        pltpu.VMEM((_ZB, _CW), jnp.float32),
        pltpu.VMEM_SHARED((acc_rows, _CW), jnp.float32),
        pltpu.SemaphoreType.DMA,
        pltpu.SemaphoreType.DMA,
    ]
    if mode in ("kg", "ex"):
        scratch.append(pltpu.VMEM((_NCH * n_wrows, _CW), jnp.float32))
    if mode == "ex":
        scratch.append(pltpu.VMEM((_SB,), jnp.float32))
        scratch.append(pltpu.VMEM((_SB,), jnp.float32))

    def body(refs):
        w_v = ev = None
        if mode == "kg":
            (src, ep_hbm, w_hbm, out, ep0, ep1, gi0, gi1, sa0, sa1, ss0, ss1,
             r0v, r1v, tmp_v, acc, semg, sems, w_v) = refs
        elif mode == "ex":
            (src, ep_hbm, w_hbm, e_hbm, out, ep0, ep1, gi0, gi1, sa0, sa1,
             ss0, ss1, r0v, r1v, tmp_v, acc, semg, sems, w_v, e0, e1) = refs
            ev = (e0, e1)
        else:
            (src, ep_hbm, out, ep0, ep1, gi0, gi1, sa0, sa1, ss0, ss1,
             r0v, r1v, tmp_v, acc, semg, sems) = refs
        ep = (ep0, ep1)
        gi = (gi0, gi1)
        sa = (sa0, sa1)
        ss = (ss0, ss1)
        rows = (r0v, r1v)

        c = lax.axis_index("c")
        s = lax.axis_index("s")
        ebase = s * pt
        mbase = s * pt * 3

        def fill_tmp(j, _):
            tmp_v[j, pl.ds(0, 16)] = jnp.zeros((16,), jnp.float32)
            tmp_v[j, pl.ds(16, 16)] = jnp.zeros((16,), jnp.float32)
            return 0

        lax.fori_loop(0, _ZB, fill_tmp, 0)
        if mode in ("kg", "ex"):
            pltpu.sync_copy(w_hbm, w_v)

        for cc in range(2):
            chunk = c * 2 + cc
            goff = chunk * n_src

            zf, zrem = zr // _ZB, zr % _ZB
            zcps = [pltpu.async_copy(
                tmp_v, acc.at[pl.ds(s * zr + zb * _ZB, _ZB)], semg)
                for zb in range(zf)]
            if zrem:
                zcps.append(pltpu.async_copy(
                    tmp_v.at[pl.ds(0, zrem)],
                    acc.at[pl.ds(s * zr + zf * _ZB, zrem)], semg))
            for cp in zcps:
                cp.wait()
            plsc.subcore_barrier()

            def load_idx(k, d):
                pltpu.sync_copy(ep_hbm.at[pl.ds(mbase + k * 3 * _SB, 3 * _SB)],
                                ep[d])
                if mode == "ex":
                    pltpu.sync_copy(e_hbm.at[pl.ds(ebase + k * _SB, _SB)], ev[d])
                for jj in range(_NI):
                    for l in range(_IB // 16):
                        fl = jj * _IB + l * 16
                        gi[d][jj, pl.ds(l * 16, 16)] = ep[d][pl.ds(fl, 16)] + goff
                        sa[d][jj, pl.ds(l * 16, 16)] = ep[d][pl.ds(_SB + fl, 16)]

            def start_gather(d):
                for j in range(_NI):
                    pltpu.async_copy(src.at[gi[d].at[j]],
                                     rows[d].at[pl.ds(j * _IB, _IB)], semg)

            def wait_gather(d):
                for j in range(_NI):
                    pltpu.make_async_copy(
                        src.at[gi[d].at[j]],
                        rows[d].at[pl.ds(j * _IB, _IB)], semg).wait()

            def issue_scatter(d):
                for jj in range(_NI):
                    for l in range(_IB // 16):
                        ss[d][jj, pl.ds(l * 16, 16)] = sa[d][jj, pl.ds(l * 16, 16)]
                for j in range(_NI):
                    pltpu.async_copy(rows[d].at[pl.ds(j * _IB, _IB)],
                                     acc.at[ss[d].at[j]], sems, add=True)

            def wait_scatter(d):
                for j in range(_NI):
                    pltpu.make_async_copy(rows[d].at[pl.ds(j * _IB, _IB)],
                                          acc.at[ss[d].at[j]], sems).wait()

            def compute(d):
                @plsc.parallel_loop(0, _SB // 16, unroll=2)
                def _group(g):
                    gb = g * 16
                    if mode in ("kg", "ex"):
                        ty16 = ep[d][pl.ds(2 * _SB + gb, 16)] + chunk * n_wrows
                    if mode == "ex":
                        sc16 = ev[d][pl.ds(gb, 16)]
                    if mode == "usr":
                        sc16 = lax.bitcast_convert_type(
                            ep[d][pl.ds(2 * _SB + gb, 16)], jnp.float32)
                    for i in range(16):
                        e = gb + i
                        r0 = rows[d][e, pl.ds(0, 16)]
                        r1 = rows[d][e, pl.ds(16, 16)]
                        if mode in ("kg", "ex"):
                            ty = ty16[i]
                            r0 = r0 * w_v[ty, pl.ds(0, 16)]
                            r1 = r1 * w_v[ty, pl.ds(16, 16)]
                        if mode in ("ex", "usr"):
                            sc = sc16[i]
                            r0 = r0 * sc
                            r1 = r1 * sc
                        rows[d][e, pl.ds(0, 16)] = r0
                        rows[d][e, pl.ds(16, 16)] = r1

            load_idx(0, 0)
            start_gather(0)

            def pair(bb, _):
                for d in (0, 1):
                    k = bb * 2 + d

                    @pl.when(k + 1 < nb2)
                    def _():
                        load_idx(k + 1, 1 - d)

                    @pl.when(k >= 1)
                    def _():
                        wait_scatter(1 - d)

                    @pl.when(k + 1 < nb2)
                    def _():
                        start_gather(1 - d)
                    wait_gather(d)
                    compute(d)
                    issue_scatter(d)
                return 0

            lax.fori_loop(0, nb2 // 2, pair, 0)
            wait_scatter(1)
            plsc.subcore_barrier()

            nblk, rem = n_dst // _ZB, n_dst % _ZB
            nper = -(-nblk // _T)

            def dblk(j, _):
                b = j * _T + s

                @pl.when(b < nblk)
                def _():
                    pltpu.sync_copy(acc.at[pl.ds(b * _ZB, _ZB)], tmp_v)
                    pltpu.sync_copy(tmp_v, out.at[pl.ds(chunk * n_dst + b * _ZB, _ZB)])
                return 0

            lax.fori_loop(0, nper, dblk, 0)
            if rem:
                @pl.when(s == 0)
                def _():
                    pltpu.sync_copy(acc.at[pl.ds(nblk * _ZB, rem)],
                                    tmp_v.at[pl.ds(0, rem)])
                    pltpu.sync_copy(tmp_v.at[pl.ds(0, rem)],
                                    out.at[pl.ds(chunk * n_dst + nblk * _ZB, rem)])
            lax.fori_loop(0, _ZB, fill_tmp, 0)
            plsc.subcore_barrier()

    def kern_fn(*refs):
        body(refs)

    return functools.partial(
        pl.kernel,
        out_type=jax.ShapeDtypeStruct((_NCH * n_dst, _CW), jnp.float32),
        mesh=plsc.VectorSubcoreMesh(**_MESH),
        scratch_types=scratch,
        compiler_params=_SC_PARAMS,
    )(kern_fn)


def _pad_to(x, n, val):
    if n > x.shape[0]:
        return jnp.pad(x, (0, n - x.shape[0]), constant_values=val)
    return x


def _padlen(e):
    q = _T * _B
    return -(-e // q) * q


def kernel(user_emb, entity_emb, interact_indices, interact_values,
           edge_index, edge_type, extra_edge_index, extra_edge_type,
           weight, extra_weight, W_Q, W_K):
    f32 = jnp.float32
    node_emb = jnp.concatenate([user_emb, entity_emb], axis=0)

    e_kg = edge_index.shape[1]
    e_ex = extra_edge_index.shape[1]
    nnz = interact_values.shape[0]
    ekg, eex, enz = _padlen(e_kg), _padlen(e_ex), _padlen(nnz)

    kg_g = _pad_to(edge_index[1].astype(jnp.int32), ekg, 0)
    kg_s = _pad_to(edge_index[0].astype(jnp.int32), ekg, _N_ENT)
    kg_t = _pad_to((edge_type - 1).astype(jnp.int32), ekg, 0)
    ex_g = _pad_to(extra_edge_index[1].astype(jnp.int32), eex, 0)
    ex_s = _pad_to(extra_edge_index[0].astype(jnp.int32), eex, _N_NODES)
    ex_h = _pad_to(extra_edge_index[0].astype(jnp.int32), eex, 0)
    ex_t = _pad_to(extra_edge_type.astype(jnp.int32), eex, 0)
    us_g = _pad_to(interact_indices[1].astype(jnp.int32), enz, 0)
    us_s = _pad_to(interact_indices[0].astype(jnp.int32), enz, _N_USERS)
    us_v = _pad_to(interact_values.astype(f32), enz, 0.0)

    wkg = weight.reshape(10, _NCH, _CW).transpose(1, 0, 2).reshape(-1, _CW)
    wex = extra_weight.reshape(8, _NCH, _CW).transpose(1, 0, 2).reshape(-1, _CW)

    def _epack(a, b, t3):
        return jnp.stack([a.reshape(-1, _SB), b.reshape(-1, _SB),
                          t3.reshape(-1, _SB)], axis=1).reshape(-1)

    kg_ep = _epack(kg_g, kg_s, kg_t)
    ex_ep = _epack(ex_g, ex_s, ex_t)
    us_ep = _epack(us_g, us_s, lax.bitcast_convert_type(us_v, jnp.int32))

    q, k = _qk(node_emb, W_Q, W_K)
    e_arr = _attn_kernel(eex)(q, k, ex_h, ex_g)
    cnt_ex, cnt_kg = _counts_kernel(eex, ekg)(ex_s, e_arr, kg_s)

    entc = _chunkify(entity_emb)
    nodc = _chunkify(node_emb)
    usrc = _chunkify(user_emb)
    ent_res, nod_res, usr_res = entc, nodc, usrc

    kg_scatter = _rowscatter_kernel(ekg, _N_ENT, _N_ENT, "kg", 10)
    ex_scatter = _rowscatter_kernel(eex, _N_NODES, _N_NODES, "ex", 8)
    us_scatter = _rowscatter_kernel(enz, _N_ENT, _N_USERS, "usr")

    ent_g = entc.reshape(_NCH * _N_ENT, _CW)
    nod_g = nodc.reshape(_NCH * _N_NODES, _CW)
    for _ in range(2):
        sums_kg = kg_scatter(ent_g, kg_ep, wkg)
        ent_mean, ent_norm, ent_res = _post(
            sums_kg.reshape(_NCH, _N_ENT, _CW), cnt_kg, ent_res,
            mean=True, attn=False, want_mean=True)
        sums_ex = ex_scatter(nod_g, ex_ep, wex, e_arr)
        _, nod_norm, nod_res = _post(
            sums_ex.reshape(_NCH, _N_NODES, _CW), cnt_ex, nod_res,
            mean=True, attn=True, want_mean=False)
        sums_us = us_scatter(ent_mean.reshape(_NCH * _N_ENT, _CW), us_ep)
        _, _, usr_res = _post(
            sums_us.reshape(_NCH, _N_USERS, _CW), None, usr_res,
            mean=False, attn=False, want_mean=False)
        ent_g = ent_norm.reshape(_NCH * _N_ENT, _CW)
        nod_g = nod_norm.reshape(_NCH * _N_NODES, _CW)

    return _unchunk(ent_res), _unchunk(usr_res), _unchunk(nod_res)


# speedup vs baseline: 1.4048x; 1.0879x over previous
import functools

import jax
import jax.numpy as jnp
from jax import lax
from jax.experimental import pallas as pl
from jax.experimental.pallas import tpu as pltpu
from jax.experimental.pallas import tpu_sc as plsc

_N_USERS = 10000
_N_ENT = 40000
_N_NODES = 50000
_C = 128
_NCH = 4
_CW = 32
_T = 16
_B = 512
_IB = 128
_SB = 256
_NI = _SB // _IB
_MESH = dict(core_axis_name="c", subcore_axis_name="s")
_SC_PARAMS = pltpu.CompilerParams(use_tc_tiling_on_sc=False)


def _qk_body(n_ref, wq_ref, wk_ref, q_ref, k_ref):
    x = n_ref[...]
    q_ref[...] = jnp.dot(x, wq_ref[...], preferred_element_type=jnp.float32)
    k_ref[...] = jnp.dot(x, wk_ref[...], preferred_element_type=jnp.float32)


def _qk(node, wq, wk):
    n = node.shape[0]
    bn = 2000
    return pl.pallas_call(
        _qk_body,
        grid=(n // bn,),
        in_specs=[
            pl.BlockSpec((bn, _C), lambda i: (i, 0)),
            pl.BlockSpec((_C, _C), lambda i: (0, 0)),
            pl.BlockSpec((_C, _C), lambda i: (0, 0)),
        ],
        out_specs=[
            pl.BlockSpec((bn, _C), lambda i: (i, 0)),
            pl.BlockSpec((bn, _C), lambda i: (i, 0)),
        ],
        out_shape=[jax.ShapeDtypeStruct((n, _C), jnp.float32)] * 2,
    )(node, wq, wk)


def _chunk_body(x_ref, o_ref):
    x = x_ref[...]
    for c in range(_NCH):
        o_ref[c] = x[:, c * _CW:(c + 1) * _CW]


def _chunkify(x):
    n = x.shape[0]
    bn = 2000
    return pl.pallas_call(
        _chunk_body,
        grid=(n // bn,),
        in_specs=[pl.BlockSpec((bn, _C), lambda i: (i, 0))],
        out_specs=pl.BlockSpec((_NCH, bn, _CW), lambda i: (0, i, 0)),
        out_shape=jax.ShapeDtypeStruct((_NCH, n, _CW), jnp.float32),
    )(x)


def _unchunk_body(x_ref, o_ref):
    for c in range(_NCH):
        o_ref[:, c * _CW:(c + 1) * _CW] = x_ref[c]


def _unchunk(x):
    n = x.shape[1]
    bn = 2000
    return pl.pallas_call(
        _unchunk_body,
        grid=(n // bn,),
        in_specs=[pl.BlockSpec((_NCH, bn, _CW), lambda i: (0, i, 0))],
        out_specs=pl.BlockSpec((bn, _C), lambda i: (i, 0)),
        out_shape=jax.ShapeDtypeStruct((n, _C), jnp.float32),
    )(x)


def _post(sums, cnt_rows, res_in, *, mean, attn, want_mean):
    n = sums.shape[1]
    bn = 2000
    grid = (n // bn,)

    def body(*refs):
        if mean:
            s_ref, c_ref, r_ref = refs[:3]
            orefs = refs[3:]
        else:
            s_ref, r_ref = refs[:2]
            orefs = refs[2:]
        s = s_ref[...]
        if mean:
            crow = c_ref[...]
            inv = 1.0 / jnp.maximum(crow[:, 0:1], 1.0)
            if attn:
                inv = inv / jnp.maximum(crow[:, 1:2], 1e-12)
            s = s * inv[None]
        nrm2 = jnp.sum(s * s, axis=(0, 2), keepdims=True)
        rinv = 1.0 / jnp.maximum(jnp.sqrt(nrm2), 1e-12)
        normed = s * rinv
        if want_mean:
            orefs[0][...] = s
            orefs = orefs[1:]
        orefs[0][...] = normed
        orefs[1][...] = r_ref[...] + normed

    in_specs = [pl.BlockSpec((_NCH, bn, _CW), lambda i: (0, i, 0))]
    args = [sums]
    if mean:
        in_specs.append(pl.BlockSpec((bn, 16), lambda i: (i, 0)))
        args.append(cnt_rows)
    in_specs.append(pl.BlockSpec((_NCH, bn, _CW), lambda i: (0, i, 0)))
    args.append(res_in)
    n_out = 3 if want_mean else 2
    out_specs = [pl.BlockSpec((_NCH, bn, _CW), lambda i: (0, i, 0))] * n_out
    out_shape = [jax.ShapeDtypeStruct((_NCH, n, _CW), jnp.float32)] * n_out
    outs = pl.pallas_call(
        body, grid=grid, in_specs=in_specs, out_specs=out_specs,
        out_shape=out_shape,
    )(*args)
    if want_mean:
        return outs[0], outs[1], outs[2]
    return None, outs[0], outs[1]


def _hsum16(v):
    for k in (8, 4, 2, 1):
        idx = jnp.bitwise_xor(lax.iota(jnp.int32, 16), k)
        v = v + v.at[idx].get(mode="promise_in_bounds")
    return v[0]

def _attn_kernel(e_pad):
    pw = e_pad // (2 * _T)
    nb = pw // _IB

    @functools.partial(
        pl.kernel,
        out_type=jax.ShapeDtypeStruct((e_pad,), jnp.float32),
        mesh=plsc.VectorSubcoreMesh(**_MESH),
        scratch_types=[
            pltpu.VMEM((2 * _IB,), jnp.int32),
            pltpu.VMEM((2 * _IB,), jnp.int32),
            pltpu.VMEM((_IB, _C), jnp.float32),
            pltpu.VMEM((_IB, _C), jnp.float32),
            pltpu.VMEM((_IB, _C), jnp.float32),
            pltpu.VMEM((_IB, _C), jnp.float32),
            pltpu.VMEM((_IB,), jnp.float32),
            pltpu.VMEM((_IB,), jnp.float32),
            pltpu.SemaphoreType.DMA,
            pltpu.SemaphoreType.DMA,
            pltpu.SemaphoreType.DMA,
            pltpu.SemaphoreType.DMA,
        ],
        compiler_params=_SC_PARAMS,
    )
    def kern(q_hbm, k_hbm, hp_hbm, e_hbm, hp0, hp1, qv0, qv1, kv0, kv1,
             ev0, ev1, semg0, semg1, semo0, semo1):
        semg = (semg0, semg1)
        semo = (semo0, semo1)
        c = lax.axis_index("c")
        s = lax.axis_index("s")
        base = (s * 2 + c) * pw
        hp = (hp0, hp1)
        qv = (qv0, qv1)
        kv = (kv0, kv1)
        ev = (ev0, ev1)

        def load_idx(k, d):
            pltpu.sync_copy(hp_hbm.at[pl.ds(base * 2 + k * 2 * _IB, 2 * _IB)],
                            hp[d])

        def start_g(d):
            pltpu.async_copy(q_hbm.at[hp[d].at[pl.ds(0, _IB)]], qv[d], semg[d])
            pltpu.async_copy(k_hbm.at[hp[d].at[pl.ds(_IB, _IB)]], kv[d], semg[d])

        def wait_g(d):
            pltpu.make_async_copy(q_hbm.at[hp[d].at[pl.ds(0, _IB)]], qv[d],
                                  semg[d]).wait()
            pltpu.make_async_copy(k_hbm.at[hp[d].at[pl.ds(_IB, _IB)]], kv[d],
                                  semg[d]).wait()

        def wait_out(d):
            pltpu.make_async_copy(ev[d], e_hbm.at[pl.ds(base, _IB)],
                                  semo[d]).wait()

        def compute(k, d):
            @plsc.parallel_loop(0, _IB // 16, unroll=2)
            def _group(g):
                def edge(i, vec):
                    e = g * 16 + i
                    acc = qv[d][e, pl.ds(0, 16)] * kv[d][e, pl.ds(0, 16)]
                    for cc in range(1, 8):
                        acc = acc + (qv[d][e, pl.ds(cc * 16, 16)]
                                     * kv[d][e, pl.ds(cc * 16, 16)])
                    sdot = _hsum16(acc)
                    return jnp.where(lax.iota(jnp.int32, 16) == i, sdot, vec)

                vec = lax.fori_loop(0, 16, edge, jnp.zeros((16,), jnp.float32))
                ev[d][pl.ds(g * 16, 16)] = jnp.exp(vec * (1.0 / 24.0))
            pltpu.async_copy(ev[d], e_hbm.at[pl.ds(base + k * _IB, _IB)], semo[d])

        load_idx(0, 0)
        start_g(0)

        def pair(bb, _):
            for d in (0, 1):
                k = bb * 2 + d

                @pl.when(k + 1 < nb)
                def _():
                    load_idx(k + 1, 1 - d)

                @pl.when(k + 1 < nb)
                def _():
                    start_g(1 - d)
                wait_g(d)

                @pl.when(k >= 2)
                def _():
                    wait_out(d)
                compute(k, d)
            return 0

        lax.fori_loop(0, nb // 2, pair, 0)
        wait_out(0)
        wait_out(1)

    return kern


def _counts_kernel(e_ex_pad, e_kg_pad):
    acc_rows = _N_NODES + 16
    nb_ex = e_ex_pad // _T // _B
    nb_kg = e_kg_pad // _T // _B

    @functools.partial(
        pl.kernel,
        out_type=(
            jax.ShapeDtypeStruct((_N_NODES, 16), jnp.float32),
            jax.ShapeDtypeStruct((_N_ENT, 16), jnp.float32),
        ),
        mesh=plsc.VectorSubcoreMesh(**_MESH),
        scratch_types=[
            pltpu.VMEM((_B,), jnp.int32),
            pltpu.VMEM((_B // _IB, _IB), jnp.int32),
            pltpu.VMEM((_B,), jnp.float32),
            pltpu.VMEM((_B, 16), jnp.float32),
            pltpu.VMEM((_IB, 16), jnp.float32),
            pltpu.VMEM_SHARED((acc_rows, 16), jnp.float32),
            pltpu.SemaphoreType.DMA,
            pltpu.SemaphoreType.DMA,
        ],
        compiler_params=_SC_PARAMS,
    )
    def kern(hh_hbm, e_hbm, h_hbm, oex_hbm, okg_hbm,
             sr_v, si_v, e_v, rows_v, tmp_v, acc, sem, sem2):
        def repack_idx():
            for jj in range(_B // _IB):
                for l in range(_IB // 16):
                    si_v[jj, pl.ds(l * 16, 16)] = sr_v[pl.ds(jj * _IB + l * 16, 16)]
        c = lax.axis_index("c")
        s = lax.axis_index("s")

        def fill_tmp(j, _):
            tmp_v[j, pl.ds(0, 16)] = jnp.zeros((16,), jnp.float32)
            return 0

        lax.fori_loop(0, _IB, fill_tmp, 0)

        zr = acc_rows // _T
        zf, zrem = zr // _IB, zr % _IB
        for zb in range(zf):
            pltpu.sync_copy(tmp_v, acc.at[pl.ds(s * zr + zb * _IB, _IB)])
        if zrem:
            pltpu.sync_copy(tmp_v.at[pl.ds(0, zrem)],
                            acc.at[pl.ds(s * zr + zf * _IB, zrem)])
        plsc.subcore_barrier()

        ones16 = jnp.where(lax.iota(jnp.int32, 16) == 0, 1.0, 0.0).astype(jnp.float32)

        def fill_ones(i, _):
            rows_v[i, pl.ds(0, 16)] = ones16
            return 0

        def ex_phase():
            pt = e_ex_pad // _T

            def batch(b, _):
                off = s * pt + b * _B
                pltpu.sync_copy(hh_hbm.at[pl.ds(off, _B)], sr_v)
                pltpu.sync_copy(e_hbm.at[pl.ds(off, _B)], e_v)
                repack_idx()

                @plsc.parallel_loop(0, _B // 16, unroll=2)
                def _group(g):
                    gb = g * 16
                    e16 = e_v[pl.ds(gb, 16)]
                    for i in range(16):
                        rows_v[gb + i, pl.ds(0, 16)] = ones16 + jnp.where(
                            lax.iota(jnp.int32, 16) == 1, e16[i], 0.0)
                cps = [
                    pltpu.async_copy(rows_v.at[pl.ds(j * _IB, _IB)],
                                     acc.at[si_v.at[j]], sem2, add=True)
                    for j in range(_B // _IB)
                ]
                for cp in cps:
                    cp.wait()
                return 0

            lax.fori_loop(0, nb_ex, batch, 0)

        def kg_phase():
            pt = e_kg_pad // _T
            lax.fori_loop(0, _B, fill_ones, 0)

            def batch(b, _):
                off = s * pt + b * _B
                pltpu.sync_copy(h_hbm.at[pl.ds(off, _B)], sr_v)
                repack_idx()
                cps = [
                    pltpu.async_copy(rows_v.at[pl.ds(j * _IB, _IB)],
                                     acc.at[si_v.at[j]], sem2, add=True)
                    for j in range(_B // _IB)
                ]
                for cp in cps:
                    cp.wait()
                return 0

            lax.fori_loop(0, nb_kg, batch, 0)

        pl.when(c == 0)(ex_phase)
        pl.when(c == 1)(kg_phase)
        plsc.subcore_barrier()

        def drain(n_dst, out_hbm):
            nblk, rem = n_dst // _IB, n_dst % _IB
            nper = -(-nblk // _T)

            def dblk(j, _):
                b = j * _T + s

                @pl.when(b < nblk)
                def _():
                    pltpu.sync_copy(acc.at[pl.ds(b * _IB, _IB)], tmp_v)
                    pltpu.sync_copy(tmp_v, out_hbm.at[pl.ds(b * _IB, _IB)])
                return 0

            lax.fori_loop(0, nper, dblk, 0)
            if rem:
                @pl.when(s == 0)
                def _():
                    pltpu.sync_copy(acc.at[pl.ds(nblk * _IB, rem)],
                                    tmp_v.at[pl.ds(0, rem)])
                    pltpu.sync_copy(tmp_v.at[pl.ds(0, rem)],
                                    out_hbm.at[pl.ds(nblk * _IB, rem)])

        pl.when(c == 0)(lambda: drain(_N_NODES, oex_hbm))
        pl.when(c == 1)(lambda: drain(_N_ENT, okg_hbm))

    return kern


def _rowscatter_kernel(e_pad, n_src, n_dst, mode, n_wrows=0):
    pt = e_pad // _T
    nb2 = pt // _SB
    assert nb2 % 2 == 0
    acc_rows = n_dst + 16
    zr = acc_rows // _T
    _ZB = 64

    scratch = [
        pltpu.VMEM((3 * _SB,), jnp.int32),
        pltpu.VMEM((3 * _SB,), jnp.int32),
        pltpu.VMEM((_NI, _IB), jnp.int32),
        pltpu.VMEM((_NI, _IB), jnp.int32),
        pltpu.VMEM((_NI, _IB), jnp.int32),
        pltpu.VMEM((_NI, _IB), jnp.int32),
        pltpu.VMEM((_NI, _IB), jnp.int32),
        pltpu.VMEM((_NI, _IB), jnp.int32),
        pltpu.VMEM((_SB, _CW), jnp.float32),
        pltpu.VMEM((_SB, _CW), jnp.float32),
        pltpu.VMEM((_ZB, _CW), jnp.float32),
        pltpu.VMEM_SHARED((acc_rows, _CW), jnp.float32),
        pltpu.SemaphoreType.DMA,
        pltpu.SemaphoreType.DMA,
        pltpu.SemaphoreType.DMA,
        pltpu.SemaphoreType.DMA,
    ]
    if mode in ("kg", "ex"):
        scratch.append(pltpu.VMEM((_NCH * n_wrows, _CW), jnp.float32))
    if mode == "ex":
        scratch.append(pltpu.VMEM((_SB,), jnp.float32))
        scratch.append(pltpu.VMEM((_SB,), jnp.float32))

    def body(refs):
        w_v = ev = None
        if mode == "kg":
            (src, ep_hbm, w_hbm, out, ep0, ep1, gi0, gi1, sa0, sa1, ss0, ss1,
             r0v, r1v, tmp_v, acc, semg0, semg1, sems0, sems1, w_v) = refs
        elif mode == "ex":
            (src, ep_hbm, w_hbm, e_hbm, out, ep0, ep1, gi0, gi1, sa0, sa1,
             ss0, ss1, r0v, r1v, tmp_v, acc, semg0, semg1, sems0, sems1,
             w_v, e0, e1) = refs
            ev = (e0, e1)
        else:
            (src, ep_hbm, out, ep0, ep1, gi0, gi1, sa0, sa1, ss0, ss1,
             r0v, r1v, tmp_v, acc, semg0, semg1, sems0, sems1) = refs
        semg = (semg0, semg1)
        sems = (sems0, sems1)
        ep = (ep0, ep1)
        gi = (gi0, gi1)
        sa = (sa0, sa1)
        ss = (ss0, ss1)
        rows = (r0v, r1v)

        c = lax.axis_index("c")
        s = lax.axis_index("s")
        ebase = s * pt
        mbase = s * pt * 3

        def fill_tmp(j, _):
            tmp_v[j, pl.ds(0, 16)] = jnp.zeros((16,), jnp.float32)
            tmp_v[j, pl.ds(16, 16)] = jnp.zeros((16,), jnp.float32)
            return 0

        lax.fori_loop(0, _ZB, fill_tmp, 0)
        if mode in ("kg", "ex"):
            pltpu.sync_copy(w_hbm, w_v)

        for cc in range(2):
            chunk = c * 2 + cc
            goff = chunk * n_src

            zf, zrem = zr // _ZB, zr % _ZB
            zcps = [pltpu.async_copy(
                tmp_v, acc.at[pl.ds(s * zr + zb * _ZB, _ZB)], semg0)
                for zb in range(zf)]
            if zrem:
                zcps.append(pltpu.async_copy(
                    tmp_v.at[pl.ds(0, zrem)],
                    acc.at[pl.ds(s * zr + zf * _ZB, zrem)], semg0))
            for cp in zcps:
                cp.wait()
            plsc.subcore_barrier()

            def load_idx(k, d):
                pltpu.sync_copy(ep_hbm.at[pl.ds(mbase + k * 3 * _SB, 3 * _SB)],
                                ep[d])
                if mode == "ex":
                    pltpu.sync_copy(e_hbm.at[pl.ds(ebase + k * _SB, _SB)], ev[d])
                for jj in range(_NI):
                    for l in range(_IB // 16):
                        fl = jj * _IB + l * 16
                        gi[d][jj, pl.ds(l * 16, 16)] = ep[d][pl.ds(fl, 16)] + goff
                        sa[d][jj, pl.ds(l * 16, 16)] = ep[d][pl.ds(_SB + fl, 16)]

            def start_gather(d):
                for j in range(_NI):
                    pltpu.async_copy(src.at[gi[d].at[j]],
                                     rows[d].at[pl.ds(j * _IB, _IB)], semg[d])

            def wait_gather(d):
                for j in range(_NI):
                    pltpu.make_async_copy(
                        src.at[gi[d].at[j]],
                        rows[d].at[pl.ds(j * _IB, _IB)], semg[d]).wait()

            def issue_scatter(d):
                for jj in range(_NI):
                    for l in range(_IB // 16):
                        ss[d][jj, pl.ds(l * 16, 16)] = sa[d][jj, pl.ds(l * 16, 16)]
                for j in range(_NI):
                    pltpu.async_copy(rows[d].at[pl.ds(j * _IB, _IB)],
                                     acc.at[ss[d].at[j]], sems[d], add=True)

            def wait_scatter(d):
                for j in range(_NI):
                    pltpu.make_async_copy(rows[d].at[pl.ds(j * _IB, _IB)],
                                          acc.at[ss[d].at[j]], sems[d]).wait()

            def compute(d):
                @plsc.parallel_loop(0, _SB // 16, unroll=2)
                def _group(g):
                    gb = g * 16
                    if mode in ("kg", "ex"):
                        ty16 = ep[d][pl.ds(2 * _SB + gb, 16)] + chunk * n_wrows
                    if mode == "ex":
                        sc16 = ev[d][pl.ds(gb, 16)]
                    if mode == "usr":
                        sc16 = lax.bitcast_convert_type(
                            ep[d][pl.ds(2 * _SB + gb, 16)], jnp.float32)
                    for i in range(16):
                        e = gb + i
                        r0 = rows[d][e, pl.ds(0, 16)]
                        r1 = rows[d][e, pl.ds(16, 16)]
                        if mode in ("kg", "ex"):
                            ty = ty16[i]
                            r0 = r0 * w_v[ty, pl.ds(0, 16)]
                            r1 = r1 * w_v[ty, pl.ds(16, 16)]
                        if mode in ("ex", "usr"):
                            sc = sc16[i]
                            r0 = r0 * sc
                            r1 = r1 * sc
                        rows[d][e, pl.ds(0, 16)] = r0
                        rows[d][e, pl.ds(16, 16)] = r1

            load_idx(0, 0)
            start_gather(0)

            def pair(bb, _):
                for d in (0, 1):
                    k = bb * 2 + d

                    @pl.when(k + 1 < nb2)
                    def _():
                        load_idx(k + 1, 1 - d)

                    @pl.when(k >= 1)
                    def _():
                        wait_scatter(1 - d)

                    @pl.when(k + 1 < nb2)
                    def _():
                        start_gather(1 - d)
                    wait_gather(d)
                    compute(d)
                    issue_scatter(d)
                return 0

            lax.fori_loop(0, nb2 // 2, pair, 0)
            wait_scatter(1)
            plsc.subcore_barrier()

            nblk, rem = n_dst // _ZB, n_dst % _ZB
            nper = -(-nblk // _T)

            def dflight(j0, _):
                cps = []
                for u in range(8):
                    b = (j0 * 8 + u) * _T + s

                    @pl.when(b < nblk)
                    def _():
                        pltpu.async_copy(
                            acc.at[pl.ds(b * _ZB, _ZB)],
                            out.at[pl.ds(chunk * n_dst + b * _ZB, _ZB)], sems0)
                for u in range(8):
                    b = (j0 * 8 + u) * _T + s

                    @pl.when(b < nblk)
                    def _():
                        pltpu.make_async_copy(
                            acc.at[pl.ds(b * _ZB, _ZB)],
                            out.at[pl.ds(chunk * n_dst + b * _ZB, _ZB)],
                            sems0).wait()
                return 0

            lax.fori_loop(0, -(-nper // 8), dflight, 0)
            if rem:
                @pl.when(s == 0)
                def _():
                    pltpu.sync_copy(acc.at[pl.ds(nblk * _ZB, rem)],
                                    out.at[pl.ds(chunk * n_dst + nblk * _ZB, rem)])
            plsc.subcore_barrier()

    def kern_fn(*refs):
        body(refs)

    return functools.partial(
        pl.kernel,
        out_type=jax.ShapeDtypeStruct((_NCH * n_dst, _CW), jnp.float32),
        mesh=plsc.VectorSubcoreMesh(**_MESH),
        scratch_types=scratch,
        compiler_params=_SC_PARAMS,
    )(kern_fn)


def _pad_to(x, n, val):
    if n > x.shape[0]:
        return jnp.pad(x, (0, n - x.shape[0]), constant_values=val)
    return x


def _padlen(e):
    q = _T * _B
    return -(-e // q) * q


def kernel(user_emb, entity_emb, interact_indices, interact_values,
           edge_index, edge_type, extra_edge_index, extra_edge_type,
           weight, extra_weight, W_Q, W_K):
    f32 = jnp.float32
    node_emb = jnp.concatenate([user_emb, entity_emb], axis=0)

    e_kg = edge_index.shape[1]
    e_ex = extra_edge_index.shape[1]
    nnz = interact_values.shape[0]
    ekg, eex, enz = _padlen(e_kg), _padlen(e_ex), _padlen(nnz)

    kg_g = _pad_to(edge_index[1].astype(jnp.int32), ekg, 0)
    kg_s = _pad_to(edge_index[0].astype(jnp.int32), ekg, _N_ENT)
    kg_t = _pad_to((edge_type - 1).astype(jnp.int32), ekg, 0)
    ex_g = _pad_to(extra_edge_index[1].astype(jnp.int32), eex, 0)
    ex_s = _pad_to(extra_edge_index[0].astype(jnp.int32), eex, _N_NODES)
    ex_h = _pad_to(extra_edge_index[0].astype(jnp.int32), eex, 0)
    ex_t = _pad_to(extra_edge_type.astype(jnp.int32), eex, 0)
    us_g = _pad_to(interact_indices[1].astype(jnp.int32), enz, 0)
    us_s = _pad_to(interact_indices[0].astype(jnp.int32), enz, _N_USERS)
    us_v = _pad_to(interact_values.astype(f32), enz, 0.0)

    wkg = weight.reshape(10, _NCH, _CW).transpose(1, 0, 2).reshape(-1, _CW)
    wex = extra_weight.reshape(8, _NCH, _CW).transpose(1, 0, 2).reshape(-1, _CW)

    def _epack(a, b, t3):
        return jnp.stack([a.reshape(-1, _SB), b.reshape(-1, _SB),
                          t3.reshape(-1, _SB)], axis=1).reshape(-1)

    kg_ep = _epack(kg_g, kg_s, kg_t)
    ex_ep = _epack(ex_g, ex_s, ex_t)
    us_ep = _epack(us_g, us_s, lax.bitcast_convert_type(us_v, jnp.int32))

    hpack = jnp.stack([ex_h.reshape(-1, _IB), ex_g.reshape(-1, _IB)],
                      axis=1).reshape(-1)
    q, k = _qk(node_emb, W_Q, W_K)
    e_arr = _attn_kernel(eex)(q, k, hpack)
    cnt_ex, cnt_kg = _counts_kernel(eex, ekg)(ex_s, e_arr, kg_s)

    entc = _chunkify(entity_emb)
    nodc = _chunkify(node_emb)
    usrc = _chunkify(user_emb)
    ent_res, nod_res, usr_res = entc, nodc, usrc

    kg_scatter = _rowscatter_kernel(ekg, _N_ENT, _N_ENT, "kg", 10)
    ex_scatter = _rowscatter_kernel(eex, _N_NODES, _N_NODES, "ex", 8)
    us_scatter = _rowscatter_kernel(enz, _N_ENT, _N_USERS, "usr")

    ent_g = entc.reshape(_NCH * _N_ENT, _CW)
    nod_g = nodc.reshape(_NCH * _N_NODES, _CW)
    for _ in range(2):
        sums_kg = kg_scatter(ent_g, kg_ep, wkg)
        ent_mean, ent_norm, ent_res = _post(
            sums_kg.reshape(_NCH, _N_ENT, _CW), cnt_kg, ent_res,
            mean=True, attn=False, want_mean=True)
        sums_ex = ex_scatter(nod_g, ex_ep, wex, e_arr)
        _, nod_norm, nod_res = _post(
            sums_ex.reshape(_NCH, _N_NODES, _CW), cnt_ex, nod_res,
            mean=True, attn=True, want_mean=False)
        sums_us = us_scatter(ent_mean.reshape(_NCH * _N_ENT, _CW), us_ep)
        _, _, usr_res = _post(
            sums_us.reshape(_NCH, _N_USERS, _CW), None, usr_res,
            mean=False, attn=False, want_mean=False)
        ent_g = ent_norm.reshape(_NCH * _N_ENT, _CW)
        nod_g = nod_norm.reshape(_NCH * _N_NODES, _CW)

    return _unchunk(ent_res), _unchunk(usr_res), _unchunk(nod_res)
